# Initial kernel scaffold; baseline (speedup 1.0000x reference)
import jax
import jax.numpy as jnp
from jax.experimental import pallas as pl


def kernel(lprobs, scores, finished):
    raise NotImplementedError("write your pallas kernel here")


# speedup vs baseline: 13.3525x; 13.3525x over previous
import functools

import jax
import jax.numpy as jnp
from jax import lax
from jax.experimental import pallas as pl
from jax.experimental.pallas import tpu as pltpu
from jax.experimental.pallas import tpu_sc as plsc

BEAM = 128
VOCAB = 32768
TOTAL = BEAM * VOCAB
NINF = float("-inf")
IMIN = -2147483648
IMAXPOS = 0x7FFFFFFF

NC = 2
NS = 16
ROWS_PER_TILE = BEAM // (NC * NS)
ELEMS_PER_TILE = ROWS_PER_TILE * VOCAB
WIN = 16384
NWIN = ELEMS_PER_TILE // WIN
VPW = WIN // 16

NB_C = 2048
CSH = 21
NB_F = 1024
FSH = 11
CAND_CAP = 8192
FINE_CAP = 256
S_CAP = 512


def _skey(v):
    b = plsc.bitcast(v, jnp.int32)
    return jnp.where(b >= 0, b, b ^ IMAXPOS)


def _prep_body(lp_ref, s_ref, f_ref, a_ref, e_ref):
    blk = lp_ref[...]
    sc = s_ref[...]
    fin = f_ref[...] > 0.0
    l0 = blk[:, 0:1]
    l1 = blk[:, 1:2]
    la = jnp.logaddexp(l1, l0)
    ninf = jnp.float32(NINF)
    a_ref[...] = jnp.broadcast_to(jnp.where(fin, ninf, sc), (BEAM, 16))
    e_ref[...] = jnp.broadcast_to(jnp.where(fin, sc, la + sc), (BEAM, 16))


def _merge_body(kr_ref, kc_ref, ir_ref, ic_ref, tv_ref, tt_ref, to_ref):
    kr = kr_ref[...]
    kc = kc_ref[...]
    ir = ir_ref[...]
    ic = ic_ref[...]
    beats = (kc > kr) | ((kc == kr) & (ic < ir))
    rank = jnp.sum(beats.astype(jnp.int32), axis=0, keepdims=True)
    p = lax.broadcasted_iota(jnp.int32, (BEAM, 2 * BEAM), 0)
    onehot = rank == p
    bits = jnp.where(kr >= 0, kr, kr ^ IMAXPOS)
    vals = lax.bitcast_convert_type(bits, jnp.float32)
    tv_ref[...] = jnp.sum(jnp.where(onehot, vals, 0.0), axis=1, keepdims=True)
    tt_ref[...] = jnp.sum(jnp.where(onehot, ir & (VOCAB - 1), 0), axis=1,
                          keepdims=True)
    to_ref[...] = jnp.sum(jnp.where(onehot, ir >> 15, 0), axis=1,
                          keepdims=True)


def _sc_body(lp_ref, a_ref, e_ref, outk_ref, outi_ref,
             buf0, buf1, hist_c, hist_f, cand_k, cand_i, fine_k, fine_i,
             merged, gbuf, chunkb, gk_vm, gi_vm, av, ev, cnt16, cnts_vm,
             lout_k, lout_i, idx128, info_vm, vok, voi, dense_k, dense_i,
             sh_hist, sh_cnts, sh_info, sh_gk, sh_gi, sh_ok, sh_oi,
             sem0, sem1):
    c = lax.axis_index("c")
    s = lax.axis_index("s")
    lane = lax.iota(jnp.int32, 16)
    zero16 = jnp.zeros((16,), jnp.int32)
    ones16 = jnp.ones((16,), jnp.int32)
    row0 = (c * NS + s) * ROWS_PER_TILE
    base_tile = row0 * VOCAB
    lanebase_c = lane * NB_C
    lanebase_f = lane * NB_F

    pltpu.sync_copy(a_ref.at[pl.ds(row0, ROWS_PER_TILE)], av)
    pltpu.sync_copy(e_ref.at[pl.ds(row0, ROWS_PER_TILE)], ev)
    for j in range(8):
        idx128[pl.ds(j * 16, 16)] = j * 16 + lane

    def _zero_range(ref, nv):
        def body(i, carry):
            ref[pl.ds(i * 16, 16)] = zero16
            return carry
        lax.fori_loop(0, nv, body, 0)

    _zero_range(hist_c, (16 * NB_C) // 16)

    def _vreg0(buf, a_sp, e_sp):
        v0 = buf[pl.ds(0, 16)] + a_sp
        v0 = jnp.where(lane == 1, e_sp, v0)
        return jnp.where((lane == 0) | (lane == 2),
                         jnp.full((16,), jnp.float32(NINF)), v0)

    for w in range(NWIN):
        buf = buf0
        pltpu.sync_copy(lp_ref.at[pl.ds(base_tile + w * WIN, WIN)], buf)
        a_sp = av[w // 2]

        def _hist_update(v):
            ub = _skey(v) ^ IMIN
            binv = lax.shift_right_logical(ub, jnp.int32(CSH))
            plsc.addupdate_scatter(hist_c, [lanebase_c + binv], ones16)

        start = 0
        if w % 2 == 0:
            _hist_update(_vreg0(buf, a_sp, ev[w // 2]))
            start = 1

        def bodyA(i, carry, buf=buf, a_sp=a_sp):
            _hist_update(buf[pl.ds(i * 16, 16)] + a_sp)
            return carry
        lax.fori_loop(start, VPW, bodyA, 0)

    def bodyM(i, carry):
        acc = hist_c[pl.ds(i * 16, 16)]
        for l in range(1, 16):
            acc = acc + hist_c[pl.ds(l * NB_C + i * 16, 16)]
        merged[pl.ds(i * 16, 16)] = acc
        return carry
    lax.fori_loop(0, NB_C // 16, bodyM, 0)
    pltpu.sync_copy(merged, sh_hist.at[s])
    plsc.subcore_barrier()

    @pl.when(s == 0)
    def _():
        pltpu.sync_copy(sh_hist.at[0], gbuf)
        for l in range(1, NS):
            pltpu.sync_copy(sh_hist.at[l], merged)

            def bodyS(i, carry):
                gbuf[pl.ds(i * 16, 16)] = (gbuf[pl.ds(i * 16, 16)]
                                           + merged[pl.ds(i * 16, 16)])
                return carry
            lax.fori_loop(0, NB_C // 16, bodyS, 0)

        def bodyT(ri, carry):
            tot, bst = carry
            r = NB_C // 16 - 1 - ri
            v = gbuf[pl.ds(r * 16, 16)]
            sfx = lax.rev(jnp.cumsum(lax.rev(v, (0,))), (0,)) + tot
            cand = jnp.where(sfx >= BEAM, r * 16 + lane, -1)
            return (tot + jnp.sum(v), jnp.maximum(bst, jnp.max(cand)))
        _, bst = lax.fori_loop(0, NB_C // 16, bodyT,
                               (jnp.int32(0), jnp.int32(-1)))
        bstar = jnp.maximum(bst, 4)
        info_vm[pl.ds(0, 16)] = jnp.where(lane == 0, jnp.full((16,), bstar),
                                          zero16)
        pltpu.sync_copy(info_vm, sh_info)
    plsc.subcore_barrier()
    pltpu.sync_copy(sh_info, info_vm)
    bstar = info_vm[pl.ds(0, 16)][0]
    keyF_ub = bstar << CSH
    skeyF = keyF_ub ^ IMIN
    tfbits = jnp.where(skeyF >= 0, skeyF, skeyF ^ IMAXPOS)
    tF = lax.bitcast_convert_type(tfbits, jnp.float32)


    _zero_range(hist_f, (16 * NB_F) // 16)
    cnt = jnp.int32(0)
    for w in range(NWIN):
        buf = buf0
        pltpu.sync_copy(lp_ref.at[pl.ds(base_tile + w * WIN, WIN)], buf)
        a_sp = av[w // 2]
        base_w = base_tile + w * WIN

        def _survivors(i, v, cc, base_w=base_w):
            sk = _skey(v)
            msk = sk >= skeyF
            ub = sk ^ IMIN
            fb = jnp.minimum(lax.shift_right_logical(ub - keyF_ub, jnp.int32(FSH)),
                             NB_F - 1)
            plsc.addupdate_scatter(hist_f, [lanebase_f + fb], ones16,
                                   mask=msk)
            c01 = jnp.where(msk, 1, 0)
            pos = jnp.clip(cc + jnp.cumsum(c01) - 1, 0, CAND_CAP - 1)
            plsc.store_scatter(cand_k, [pos], sk, mask=msk)
            plsc.store_scatter(cand_i, [pos], base_w + i * 16 + lane,
                               mask=msk)
            return cc + jnp.sum(c01)

        start = 0
        if w % 2 == 0:
            v0 = _vreg0(buf, a_sp, ev[w // 2])
            cnt = _survivors(0, v0, cnt)
            start = 1

        def bodyB(i, cc, buf=buf, a_sp=a_sp):
            v = buf[pl.ds(i * 16, 16)] + a_sp
            return _survivors(i, v, cc)
        cnt = lax.fori_loop(start, VPW, bodyB, cnt)

    def bodyM2(i, carry):
        acc = hist_f[pl.ds(i * 16, 16)]
        for l in range(1, 16):
            acc = acc + hist_f[pl.ds(l * NB_F + i * 16, 16)]
        merged[pl.ds(i * 16, 16)] = acc
        return carry
    lax.fori_loop(0, NB_F // 16, bodyM2, 0)
    pltpu.sync_copy(merged.at[pl.ds(0, NB_F)], sh_hist.at[s, pl.ds(0, NB_F)])
    plsc.subcore_barrier()

    @pl.when(s == 0)
    def _():
        pltpu.sync_copy(sh_hist.at[0, pl.ds(0, NB_F)], gbuf.at[pl.ds(0, NB_F)])
        for l in range(1, NS):
            pltpu.sync_copy(sh_hist.at[l, pl.ds(0, NB_F)],
                            merged.at[pl.ds(0, NB_F)])

            def bodyS(i, carry):
                gbuf[pl.ds(i * 16, 16)] = (gbuf[pl.ds(i * 16, 16)]
                                           + merged[pl.ds(i * 16, 16)])
                return carry
            lax.fori_loop(0, NB_F // 16, bodyS, 0)

        def bodyT(ri, carry):
            tot, bst = carry
            r = NB_F // 16 - 1 - ri
            v = gbuf[pl.ds(r * 16, 16)]
            sfx = lax.rev(jnp.cumsum(lax.rev(v, (0,))), (0,)) + tot
            cand = jnp.where(sfx >= BEAM, r * 16 + lane, -1)
            return (tot + jnp.sum(v), jnp.maximum(bst, jnp.max(cand)))
        _, fbst = lax.fori_loop(0, NB_F // 16, bodyT,
                                (jnp.int32(0), jnp.int32(-1)))
        fb = jnp.maximum(fbst, 0)
        skeyT = (keyF_ub + (fb << FSH)) ^ IMIN
        info_vm[pl.ds(0, 16)] = jnp.where(lane == 0, jnp.full((16,), skeyT),
                                          zero16)
        pltpu.sync_copy(info_vm, sh_info)
    plsc.subcore_barrier()
    pltpu.sync_copy(sh_info, info_vm)
    skeyT = info_vm[pl.ds(0, 16)][0]


    for j in range(FINE_CAP // 16):
        fine_k[pl.ds(j * 16, 16)] = jnp.full((16,), IMIN, jnp.int32)
        fine_i[pl.ds(j * 16, 16)] = (0x40000000 + s * FINE_CAP
                                     + j * 16 + lane)

    cmin = jnp.minimum(cnt, CAND_CAP)

    def bodyF(j, fc):
        kv = cand_k[pl.ds(j * 16, 16)]
        iv = cand_i[pl.ds(j * 16, 16)]
        msk = ((j * 16 + lane) < cmin) & (kv >= skeyT)
        c01 = jnp.where(msk, 1, 0)
        pos = jnp.clip(fc + jnp.cumsum(c01) - 1, 0, FINE_CAP - 1)
        plsc.store_scatter(fine_k, [pos], kv, mask=msk)
        plsc.store_scatter(fine_i, [pos], iv, mask=msk)
        return fc + jnp.sum(c01)
    fcnt = lax.fori_loop(0, CAND_CAP // 16, bodyF, jnp.int32(0))
    fpad = (jnp.minimum(fcnt, FINE_CAP) + 15) // 16 * 16

    cnt16[pl.ds(0, 16)] = jnp.full((16,), fpad)
    pltpu.sync_copy(cnt16, sh_cnts.at[s])

    plsc.subcore_barrier()

    pltpu.sync_copy(sh_cnts, cnts_vm)

    my_off = jnp.int32(0)
    for t in range(NS):
        v = cnts_vm[t]
        my_off = my_off + jnp.where(t < s, v[0], 0)


    for j in range(S_CAP // 16):
        dense_k[pl.ds(j * 16, 16)] = zero16
        dense_i[pl.ds(j * 16, 16)] = zero16
    for j in range(FINE_CAP // 16):
        kv = fine_k[pl.ds(j * 16, 16)]
        iv = fine_i[pl.ds(j * 16, 16)]
        pmask = (j * 16 + lane) < fpad
        pos2 = jnp.clip(my_off + j * 16 + lane, 0, S_CAP - 1)
        plsc.store_scatter(dense_k, [pos2], kv ^ IMIN, mask=pmask)
        plsc.store_scatter(dense_i, [pos2], iv, mask=pmask)
    pltpu.sync_copy(dense_k, sh_gk.at[s])
    pltpu.sync_copy(dense_i, sh_gi.at[s])
    plsc.subcore_barrier()

    for j in range(S_CAP // 16):
        gk_vm[pl.ds(j * 16, 16)] = zero16
        gi_vm[pl.ds(j * 16, 16)] = zero16
    for l in range(NS):
        pltpu.sync_copy(sh_gk.at[l], dense_k)
        pltpu.sync_copy(sh_gi.at[l], dense_i)

        def bodyAcc(j, carry):
            gk_vm[pl.ds(j * 16, 16)] = (gk_vm[pl.ds(j * 16, 16)]
                                        + dense_k[pl.ds(j * 16, 16)])
            gi_vm[pl.ds(j * 16, 16)] = (gi_vm[pl.ds(j * 16, 16)]
                                        + dense_i[pl.ds(j * 16, 16)])
            return carry
        lax.fori_loop(0, S_CAP // 16, bodyAcc, 0)

    def bodyPad(j, carry):
        u = gk_vm[pl.ds(j * 16, 16)]
        empty = u == 0
        gk_vm[pl.ds(j * 16, 16)] = jnp.where(
            empty, jnp.full((16,), IMIN, jnp.int32), u ^ IMIN)
        iv = gi_vm[pl.ds(j * 16, 16)]
        gi_vm[pl.ds(j * 16, 16)] = jnp.where(
            empty, 0x50000000 + j * 16 + lane, iv)
        return carry
    lax.fori_loop(0, S_CAP // 16, bodyPad, 0)
    for j in range(8):
        lout_k[pl.ds(j * 16, 16)] = zero16
        lout_i[pl.ds(j * 16, 16)] = zero16

    for g in range((S_CAP // NS) // 16):
        ke_vec = gk_vm[pl.ds(s * (S_CAP // NS) + g * 16, 16)]
        ie_vec = gi_vm[pl.ds(s * (S_CAP // NS) + g * 16, 16)]

        def bodyRR(j, acc, ke_vec=ke_vec, ie_vec=ie_vec):
            kv = gk_vm[pl.ds(j * 16, 16)]
            iv = gi_vm[pl.ds(j * 16, 16)]
            for t in range(16):
                kb = jnp.full((16,), kv[t])
                ib = jnp.full((16,), iv[t])
                beats = (kb > ke_vec) | ((kb == ke_vec) & (ib < ie_vec))
                acc = acc + jnp.where(beats, 1, 0)
            return acc
        rank_vec = lax.fori_loop(0, S_CAP // 16, bodyRR, zero16)
        msk = rank_vec < BEAM
        plsc.store_scatter(lout_k, [rank_vec], ke_vec, mask=msk)
        plsc.store_scatter(lout_i, [rank_vec], ie_vec, mask=msk)
    pltpu.sync_copy(lout_k, sh_ok.at[s])
    pltpu.sync_copy(lout_i, sh_oi.at[s])
    plsc.subcore_barrier()

    @pl.when(s == 0)
    def _():
        pltpu.sync_copy(sh_ok, vok)
        pltpu.sync_copy(sh_oi, voi)
        for g in range(8):
            acck = vok[0, pl.ds(g * 16, 16)]
            acci = voi[0, pl.ds(g * 16, 16)]
            for l in range(1, 16):
                acck = acck + vok[l, pl.ds(g * 16, 16)]
                acci = acci + voi[l, pl.ds(g * 16, 16)]
            lout_k[pl.ds(g * 16, 16)] = acck
            lout_i[pl.ds(g * 16, 16)] = acci
        pltpu.sync_copy(lout_k, outk_ref.at[c])
        pltpu.sync_copy(lout_i, outi_ref.at[c])


_prep_call = pl.pallas_call(
    _prep_body,
    out_shape=[
        jax.ShapeDtypeStruct((BEAM, 16), jnp.float32),
        jax.ShapeDtypeStruct((BEAM, 16), jnp.float32),
    ],
)

_merge_call = pl.pallas_call(
    _merge_body,
    out_shape=[
        jax.ShapeDtypeStruct((BEAM, 1), jnp.float32),
        jax.ShapeDtypeStruct((BEAM, 1), jnp.int32),
        jax.ShapeDtypeStruct((BEAM, 1), jnp.int32),
    ],
)

@functools.cache
def _make_sc_call():
  return functools.partial(
    pl.kernel,
    out_type=(
        jax.ShapeDtypeStruct((NC, BEAM), jnp.int32),
        jax.ShapeDtypeStruct((NC, BEAM), jnp.int32),
    ),
    mesh=plsc.VectorSubcoreMesh(core_axis_name="c", subcore_axis_name="s",
                                num_cores=NC, num_subcores=NS),
    compiler_params=pltpu.CompilerParams(needs_layout_passes=False),
    scratch_types=[
        pltpu.VMEM((WIN,), jnp.float32),
        pltpu.VMEM((WIN,), jnp.float32),
        pltpu.VMEM((16 * NB_C,), jnp.int32),
        pltpu.VMEM((16 * NB_F,), jnp.int32),
        pltpu.VMEM((CAND_CAP,), jnp.int32),
        pltpu.VMEM((CAND_CAP,), jnp.int32),
        pltpu.VMEM((FINE_CAP,), jnp.int32),
        pltpu.VMEM((FINE_CAP,), jnp.int32),
        pltpu.VMEM((NB_C,), jnp.int32),
        pltpu.VMEM((NB_C,), jnp.int32),
        pltpu.VMEM((16, 256), jnp.int32),
        pltpu.VMEM((S_CAP,), jnp.int32),
        pltpu.VMEM((S_CAP,), jnp.int32),
        pltpu.VMEM((ROWS_PER_TILE, 16), jnp.float32),
        pltpu.VMEM((ROWS_PER_TILE, 16), jnp.float32),
        pltpu.VMEM((16,), jnp.int32),
        pltpu.VMEM((NS, 16), jnp.int32),
        pltpu.VMEM((BEAM,), jnp.int32),
        pltpu.VMEM((BEAM,), jnp.int32),
        pltpu.VMEM((BEAM,), jnp.int32),
        pltpu.VMEM((16,), jnp.int32),
        pltpu.VMEM((NS, BEAM), jnp.int32),
        pltpu.VMEM((NS, BEAM), jnp.int32),
        pltpu.VMEM((S_CAP,), jnp.int32),
        pltpu.VMEM((S_CAP,), jnp.int32),
        pltpu.VMEM_SHARED((NS, NB_C), jnp.int32),
        pltpu.VMEM_SHARED((NS, 16), jnp.int32),
        pltpu.VMEM_SHARED((16,), jnp.int32),
        pltpu.VMEM_SHARED((NS, S_CAP), jnp.int32),
        pltpu.VMEM_SHARED((NS, S_CAP), jnp.int32),
        pltpu.VMEM_SHARED((NS, BEAM), jnp.int32),
        pltpu.VMEM_SHARED((NS, BEAM), jnp.int32),
        pltpu.SemaphoreType.DMA,
        pltpu.SemaphoreType.DMA,
    ],
  )(_sc_body)


def kernel(lprobs, scores, finished):
    lp1 = lprobs.reshape(TOTAL)
    finf = finished.astype(jnp.float32).reshape(BEAM, 1)
    a2, e2 = _prep_call(lprobs[:, :128], scores, finf)
    sck, sci = _make_sc_call()(lp1, a2, e2)
    kr = sck.reshape(1, 2 * BEAM)
    kc = sck.reshape(2 * BEAM, 1)
    ir = sci.reshape(1, 2 * BEAM)
    ic = sci.reshape(2 * BEAM, 1)
    tv, tt, to = _merge_call(kr, kc, ir, ic)
    return tv.reshape(BEAM), tt.reshape(BEAM), to.reshape(BEAM)
